# baseline (speedup 1.0000x reference)
import functools

import jax
import jax.numpy as jnp
from jax import lax
from jax.experimental import pallas as pl
from jax.experimental.pallas import tpu as pltpu
from jax.experimental.pallas import tpu_sc as plsc

HIDDEN = 64
CHUNK = 128
TBLOCK = 512


def _fuse_body(a_ref, b_ref, out_ref):
    out_ref[...] = jnp.concatenate(
        [a_ref[...].T, b_ref[...].T], axis=1)


def _fused_node_table(node_t, node_im_t):
    d, n = node_t.shape
    grid = (n + TBLOCK - 1) // TBLOCK
    return pl.pallas_call(
        _fuse_body,
        grid=(grid,),
        in_specs=[
            pl.BlockSpec((d, TBLOCK), lambda i: (0, i)),
            pl.BlockSpec((d, TBLOCK), lambda i: (0, i)),
        ],
        out_specs=pl.BlockSpec((TBLOCK, 2 * d), lambda i: (i, 0)),
        out_shape=jax.ShapeDtypeStruct((n, 2 * d), jnp.float32),
    )(node_t, node_im_t)


def _trig_body(rel_ref, cs_ref):
    theta = rel_ref[...]
    cs_ref[:, :HIDDEN] = jnp.cos(theta)
    cs_ref[:, HIDDEN:] = jnp.sin(theta)


def _trig_table(rel_emb):
    r, d = rel_emb.shape
    return pl.pallas_call(
        _trig_body,
        out_shape=jax.ShapeDtypeStruct((r, 2 * d), rel_emb.dtype),
    )(rel_emb)


def _make_sc_gather(batch, d2, nw):
    b_per_w = batch // nw
    n_chunks = b_per_w // CHUNK
    mesh = plsc.VectorSubcoreMesh(core_axis_name="c", subcore_axis_name="s")
    out_sds = jax.ShapeDtypeStruct((batch, d2), jnp.float32)

    @functools.partial(
        pl.kernel,
        mesh=mesh,
        out_type=(out_sds, out_sds, out_sds),
        scratch_types=[
            pltpu.VMEM((n_chunks, CHUNK), jnp.int32),
            pltpu.VMEM((n_chunks, CHUNK), jnp.int32),
            pltpu.VMEM((n_chunks, CHUNK), jnp.int32),
            pltpu.VMEM((CHUNK, d2), jnp.float32),
            pltpu.VMEM((CHUNK, d2), jnp.float32),
            pltpu.SemaphoreType.DMA,
            pltpu.SemaphoreType.DMA,
            pltpu.SemaphoreType.DMA,
            pltpu.SemaphoreType.DMA,
        ],
    )
    def sc_gather(h_idx, r_idx, t_idx, t_node, t_cs,
                  o_h, o_r, o_t,
                  hv, rv, tv, buf0, buf1, sg0, sg1, sw0, sw1):
        nc = 2
        wid = lax.axis_index("s") * nc + lax.axis_index("c")
        base = wid * b_per_w
        pltpu.sync_copy(h_idx.at[wid], hv)
        pltpu.sync_copy(r_idx.at[wid], rv)
        pltpu.sync_copy(t_idx.at[wid], tv)

        bufs = (buf0, buf1)
        sgs = (sg0, sg1)
        sws = (sw0, sw1)
        jobs = [(tab, idx, out, c)
                for (tab, idx, out) in ((t_node, hv, o_h), (t_cs, rv, o_r),
                                        (t_node, tv, o_t))
                for c in range(n_chunks)]
        nj = len(jobs)
        g_wait = [None] * nj
        w_wait = [None] * nj

        def start_gather(k):
            tab, idx, _out, c = jobs[k]
            g_wait[k] = pltpu.async_copy(
                tab.at[idx.at[c]], bufs[k % 2], sgs[k % 2])

        def start_write(k):
            _tab, _idx, out, c = jobs[k]
            w_wait[k] = pltpu.async_copy(
                bufs[k % 2], out.at[pl.ds(base + c * CHUNK, CHUNK)],
                sws[k % 2])

        start_gather(0)
        for k in range(nj):
            if k + 1 < nj:
                if k >= 1:
                    w_wait[k - 1].wait()
                start_gather(k + 1)
            g_wait[k].wait()
            start_write(k)
        w_wait[nj - 2].wait()
        w_wait[nj - 1].wait()

    return sc_gather


def kernel(head_index, rel_type, tail_index, node_emb, node_emb_im, rel_emb):
    batch = head_index.shape[0]
    d = node_emb.shape[1]
    info = plsc.get_sparse_core_info()
    nw = info.num_cores * info.num_subcores
    b_per_w = batch // nw
    n_chunks = b_per_w // CHUNK

    node_cs = _fused_node_table(node_emb.T, node_emb_im.T)
    rel_cs = _trig_table(rel_emb)

    h_idx = head_index.astype(jnp.int32).reshape(nw, n_chunks, CHUNK)
    r_idx = rel_type.astype(jnp.int32).reshape(nw, n_chunks, CHUNK)
    t_idx = tail_index.astype(jnp.int32).reshape(nw, n_chunks, CHUNK)

    sc_gather = _make_sc_gather(batch, 2 * d, nw)
    o_h, o_r, o_t = sc_gather(h_idx, r_idx, t_idx, node_cs, rel_cs)
    return (o_h[:, :d], o_h[:, d:], o_r[:, :d], o_r[:, d:],
            o_t[:, :d], o_t[:, d:])


# speedup vs baseline: 2.2894x; 2.2894x over previous
import functools

import jax
import jax.numpy as jnp
from jax import lax
from jax.experimental import pallas as pl
from jax.experimental.pallas import tpu as pltpu
from jax.experimental.pallas import tpu_sc as plsc

HIDDEN = 64
CHUNK = 128
TBLOCK = 2048


def _fuse_body(a_ref, b_ref, out_ref):
    d2 = 2 * a_ref.shape[0]
    eye = jnp.asarray(
        jax.lax.broadcasted_iota(jnp.int32, (d2, d2), 0)
        == jax.lax.broadcasted_iota(jnp.int32, (d2, d2), 1),
        dtype=jnp.float32)
    ab = jnp.concatenate([a_ref[...], b_ref[...]], axis=0)
    out_ref[...] = jax.lax.dot_general(
        ab, eye, (((0,), (0,)), ((), ())),
        preferred_element_type=jnp.float32)


def _fused_node_table(node_t, node_im_t):
    d, n = node_t.shape
    grid = (n + TBLOCK - 1) // TBLOCK
    return pl.pallas_call(
        _fuse_body,
        grid=(grid,),
        in_specs=[
            pl.BlockSpec((d, TBLOCK), lambda i: (0, i)),
            pl.BlockSpec((d, TBLOCK), lambda i: (0, i)),
        ],
        out_specs=pl.BlockSpec((TBLOCK, 2 * d), lambda i: (i, 0)),
        out_shape=jax.ShapeDtypeStruct((n, 2 * d), jnp.float32),
    )(node_t, node_im_t)


def _trig_body(rel_ref, cs_ref):
    theta = rel_ref[...]
    cs_ref[:, :HIDDEN] = jnp.cos(theta)
    cs_ref[:, HIDDEN:] = jnp.sin(theta)


def _trig_table(rel_emb):
    r, d = rel_emb.shape
    return pl.pallas_call(
        _trig_body,
        out_shape=jax.ShapeDtypeStruct((r, 2 * d), rel_emb.dtype),
    )(rel_emb)


def _make_sc_gather(batch, d2, nw):
    b_per_w = batch // nw
    n_chunks = b_per_w // CHUNK
    mesh = plsc.VectorSubcoreMesh(core_axis_name="c", subcore_axis_name="s")
    out_sds = jax.ShapeDtypeStruct((batch, d2), jnp.float32)

    @functools.partial(
        pl.kernel,
        mesh=mesh,
        out_type=(out_sds, out_sds, out_sds),
        scratch_types=[
            pltpu.VMEM((n_chunks, CHUNK), jnp.int32),
            pltpu.VMEM((n_chunks, CHUNK), jnp.int32),
            pltpu.VMEM((n_chunks, CHUNK), jnp.int32),
            pltpu.VMEM((CHUNK, d2), jnp.float32),
            pltpu.VMEM((CHUNK, d2), jnp.float32),
            pltpu.SemaphoreType.DMA,
            pltpu.SemaphoreType.DMA,
            pltpu.SemaphoreType.DMA,
            pltpu.SemaphoreType.DMA,
        ],
    )
    def sc_gather(h_idx, r_idx, t_idx, t_node, t_cs,
                  o_h, o_r, o_t,
                  hv, rv, tv, buf0, buf1, sg0, sg1, sw0, sw1):
        nc = 2
        wid = lax.axis_index("s") * nc + lax.axis_index("c")
        base = wid * b_per_w
        pltpu.sync_copy(h_idx.at[wid], hv)
        pltpu.sync_copy(r_idx.at[wid], rv)
        pltpu.sync_copy(t_idx.at[wid], tv)

        bufs = (buf0, buf1)
        sgs = (sg0, sg1)
        sws = (sw0, sw1)
        jobs = [(tab, idx, out, c)
                for (tab, idx, out) in ((t_node, hv, o_h), (t_cs, rv, o_r),
                                        (t_node, tv, o_t))
                for c in range(n_chunks)]
        nj = len(jobs)
        g_wait = [None] * nj
        w_wait = [None] * nj

        def start_gather(k):
            tab, idx, _out, c = jobs[k]
            g_wait[k] = pltpu.async_copy(
                tab.at[idx.at[c]], bufs[k % 2], sgs[k % 2])

        def start_write(k):
            _tab, _idx, out, c = jobs[k]
            w_wait[k] = pltpu.async_copy(
                bufs[k % 2], out.at[pl.ds(base + c * CHUNK, CHUNK)],
                sws[k % 2])

        start_gather(0)
        for k in range(nj):
            if k + 1 < nj:
                if k >= 1:
                    w_wait[k - 1].wait()
                start_gather(k + 1)
            g_wait[k].wait()
            start_write(k)
        w_wait[nj - 2].wait()
        w_wait[nj - 1].wait()

    return sc_gather


def kernel(head_index, rel_type, tail_index, node_emb, node_emb_im, rel_emb):
    batch = head_index.shape[0]
    d = node_emb.shape[1]
    info = plsc.get_sparse_core_info()
    nw = info.num_cores * info.num_subcores
    b_per_w = batch // nw
    n_chunks = b_per_w // CHUNK

    node_cs = _fused_node_table(node_emb.T, node_emb_im.T)
    rel_cs = _trig_table(rel_emb)

    h_idx = head_index.astype(jnp.int32).reshape(nw, n_chunks, CHUNK)
    r_idx = rel_type.astype(jnp.int32).reshape(nw, n_chunks, CHUNK)
    t_idx = tail_index.astype(jnp.int32).reshape(nw, n_chunks, CHUNK)

    sc_gather = _make_sc_gather(batch, 2 * d, nw)
    o_h, o_r, o_t = sc_gather(h_idx, r_idx, t_idx, node_cs, rel_cs)
    return (o_h[:, :d], o_h[:, d:], o_r[:, :d], o_r[:, d:],
            o_t[:, :d], o_t[:, d:])


# speedup vs baseline: 3.3680x; 1.4711x over previous
import functools

import jax
import jax.numpy as jnp
from jax import lax
from jax.experimental import pallas as pl
from jax.experimental.pallas import tpu as pltpu
from jax.experimental.pallas import tpu_sc as plsc

HIDDEN = 64
CHUNK = 128
TBLOCK = 8192


def _fuse_body(a_ref, b_ref, out_ref):
    d2 = 2 * a_ref.shape[0]
    eye = jnp.asarray(
        jax.lax.broadcasted_iota(jnp.int32, (d2, d2), 0)
        == jax.lax.broadcasted_iota(jnp.int32, (d2, d2), 1),
        dtype=jnp.float32)
    ab = jnp.concatenate([a_ref[...], b_ref[...]], axis=0)
    out_ref[...] = jax.lax.dot_general(
        ab, eye, (((0,), (0,)), ((), ())),
        preferred_element_type=jnp.float32)


def _fused_node_table(node_t, node_im_t):
    d, n = node_t.shape
    grid = (n + TBLOCK - 1) // TBLOCK
    return pl.pallas_call(
        _fuse_body,
        grid=(grid,),
        in_specs=[
            pl.BlockSpec((d, TBLOCK), lambda i: (0, i)),
            pl.BlockSpec((d, TBLOCK), lambda i: (0, i)),
        ],
        out_specs=pl.BlockSpec((TBLOCK, 2 * d), lambda i: (i, 0)),
        out_shape=jax.ShapeDtypeStruct((n, 2 * d), jnp.float32),
    )(node_t, node_im_t)


def _trig_body(rel_ref, cs_ref):
    theta = rel_ref[...]
    cs_ref[:, :HIDDEN] = jnp.cos(theta)
    cs_ref[:, HIDDEN:] = jnp.sin(theta)


def _trig_table(rel_emb):
    r, d = rel_emb.shape
    return pl.pallas_call(
        _trig_body,
        out_shape=jax.ShapeDtypeStruct((r, 2 * d), rel_emb.dtype),
    )(rel_emb)


def _make_sc_gather(batch, d2, nw):
    b_per_w = batch // nw
    n_chunks = b_per_w // CHUNK
    mesh = plsc.VectorSubcoreMesh(core_axis_name="c", subcore_axis_name="s")
    out_sds = jax.ShapeDtypeStruct((batch, d2), jnp.float32)

    @functools.partial(
        pl.kernel,
        mesh=mesh,
        out_type=(out_sds, out_sds, out_sds),
        scratch_types=[
            pltpu.VMEM((n_chunks, CHUNK), jnp.int32),
            pltpu.VMEM((n_chunks, CHUNK), jnp.int32),
            pltpu.VMEM((n_chunks, CHUNK), jnp.int32),
            pltpu.VMEM((CHUNK, d2), jnp.float32),
            pltpu.VMEM((CHUNK, d2), jnp.float32),
            pltpu.SemaphoreType.DMA,
            pltpu.SemaphoreType.DMA,
            pltpu.SemaphoreType.DMA,
            pltpu.SemaphoreType.DMA,
        ],
    )
    def sc_gather(h_idx, r_idx, t_idx, t_node, t_cs,
                  o_h, o_r, o_t,
                  hv, rv, tv, buf0, buf1, sg0, sg1, sw0, sw1):
        nc = 2
        wid = lax.axis_index("s") * nc + lax.axis_index("c")
        base = wid * b_per_w
        pltpu.sync_copy(h_idx.at[wid], hv)
        pltpu.sync_copy(r_idx.at[wid], rv)
        pltpu.sync_copy(t_idx.at[wid], tv)

        bufs = (buf0, buf1)
        sgs = (sg0, sg1)
        sws = (sw0, sw1)
        jobs = [(tab, idx, out, c)
                for (tab, idx, out) in ((t_node, hv, o_h), (t_cs, rv, o_r),
                                        (t_node, tv, o_t))
                for c in range(n_chunks)]
        nj = len(jobs)
        g_wait = [None] * nj
        w_wait = [None] * nj

        def start_gather(k):
            tab, idx, _out, c = jobs[k]
            g_wait[k] = pltpu.async_copy(
                tab.at[idx.at[c]], bufs[k % 2], sgs[k % 2])

        def start_write(k):
            _tab, _idx, out, c = jobs[k]
            w_wait[k] = pltpu.async_copy(
                bufs[k % 2], out.at[pl.ds(base + c * CHUNK, CHUNK)],
                sws[k % 2])

        start_gather(0)
        for k in range(nj):
            if k + 1 < nj:
                if k >= 1:
                    w_wait[k - 1].wait()
                start_gather(k + 1)
            g_wait[k].wait()
            start_write(k)
        w_wait[nj - 2].wait()
        w_wait[nj - 1].wait()

    return sc_gather


def kernel(head_index, rel_type, tail_index, node_emb, node_emb_im, rel_emb):
    batch = head_index.shape[0]
    d = node_emb.shape[1]
    info = plsc.get_sparse_core_info()
    nw = info.num_cores * info.num_subcores
    b_per_w = batch // nw
    n_chunks = b_per_w // CHUNK

    node_cs = _fused_node_table(node_emb.T, node_emb_im.T)
    rel_cs = _trig_table(rel_emb)

    h_idx = head_index.astype(jnp.int32).reshape(nw, n_chunks, CHUNK)
    r_idx = rel_type.astype(jnp.int32).reshape(nw, n_chunks, CHUNK)
    t_idx = tail_index.astype(jnp.int32).reshape(nw, n_chunks, CHUNK)

    sc_gather = _make_sc_gather(batch, 2 * d, nw)
    o_h, o_r, o_t = sc_gather(h_idx, r_idx, t_idx, node_cs, rel_cs)
    return (o_h[:, :d], o_h[:, d:], o_r[:, :d], o_r[:, d:],
            o_t[:, :d], o_t[:, d:])


# speedup vs baseline: 3.4167x; 1.0145x over previous
import functools

import jax
import jax.numpy as jnp
from jax import lax
from jax.experimental import pallas as pl
from jax.experimental.pallas import tpu as pltpu
from jax.experimental.pallas import tpu_sc as plsc

HIDDEN = 64
CHUNK = 128
TBLOCK = 16384


def _fuse_body(a_ref, b_ref, out_ref):
    d2 = 2 * a_ref.shape[0]
    eye = jnp.asarray(
        jax.lax.broadcasted_iota(jnp.int32, (d2, d2), 0)
        == jax.lax.broadcasted_iota(jnp.int32, (d2, d2), 1),
        dtype=jnp.float32)
    ab = jnp.concatenate([a_ref[...], b_ref[...]], axis=0)
    out_ref[...] = jax.lax.dot_general(
        ab, eye, (((0,), (0,)), ((), ())),
        preferred_element_type=jnp.float32)


def _fused_node_table(node_t, node_im_t):
    d, n = node_t.shape
    grid = (n + TBLOCK - 1) // TBLOCK
    return pl.pallas_call(
        _fuse_body,
        grid=(grid,),
        in_specs=[
            pl.BlockSpec((d, TBLOCK), lambda i: (0, i)),
            pl.BlockSpec((d, TBLOCK), lambda i: (0, i)),
        ],
        out_specs=pl.BlockSpec((TBLOCK, 2 * d), lambda i: (i, 0)),
        out_shape=jax.ShapeDtypeStruct((n, 2 * d), jnp.float32),
    )(node_t, node_im_t)


def _trig_body(rel_ref, cs_ref):
    theta = rel_ref[...]
    cs_ref[:, :HIDDEN] = jnp.cos(theta)
    cs_ref[:, HIDDEN:] = jnp.sin(theta)


def _trig_table(rel_emb):
    r, d = rel_emb.shape
    return pl.pallas_call(
        _trig_body,
        out_shape=jax.ShapeDtypeStruct((r, 2 * d), rel_emb.dtype),
    )(rel_emb)


def _make_sc_gather(batch, d2, nw):
    b_per_w = batch // nw
    n_chunks = b_per_w // CHUNK
    mesh = plsc.VectorSubcoreMesh(core_axis_name="c", subcore_axis_name="s")
    out_sds = jax.ShapeDtypeStruct((batch, d2), jnp.float32)

    @functools.partial(
        pl.kernel,
        mesh=mesh,
        out_type=(out_sds, out_sds, out_sds),
        scratch_types=[
            pltpu.VMEM((n_chunks, CHUNK), jnp.int32),
            pltpu.VMEM((n_chunks, CHUNK), jnp.int32),
            pltpu.VMEM((n_chunks, CHUNK), jnp.int32),
            pltpu.VMEM((CHUNK, d2), jnp.float32),
            pltpu.VMEM((CHUNK, d2), jnp.float32),
            pltpu.SemaphoreType.DMA,
            pltpu.SemaphoreType.DMA,
            pltpu.SemaphoreType.DMA,
            pltpu.SemaphoreType.DMA,
        ],
    )
    def sc_gather(h_idx, r_idx, t_idx, t_node, t_cs,
                  o_h, o_r, o_t,
                  hv, rv, tv, buf0, buf1, sg0, sg1, sw0, sw1):
        nc = 2
        wid = lax.axis_index("s") * nc + lax.axis_index("c")
        base = wid * b_per_w
        pltpu.sync_copy(h_idx.at[wid], hv)
        pltpu.sync_copy(r_idx.at[wid], rv)
        pltpu.sync_copy(t_idx.at[wid], tv)

        bufs = (buf0, buf1)
        sgs = (sg0, sg1)
        sws = (sw0, sw1)
        jobs = [(tab, idx, out, c)
                for (tab, idx, out) in ((t_node, hv, o_h), (t_cs, rv, o_r),
                                        (t_node, tv, o_t))
                for c in range(n_chunks)]
        nj = len(jobs)
        g_wait = [None] * nj
        w_wait = [None] * nj

        def start_gather(k):
            tab, idx, _out, c = jobs[k]
            g_wait[k] = pltpu.async_copy(
                tab.at[idx.at[c]], bufs[k % 2], sgs[k % 2])

        def start_write(k):
            _tab, _idx, out, c = jobs[k]
            w_wait[k] = pltpu.async_copy(
                bufs[k % 2], out.at[pl.ds(base + c * CHUNK, CHUNK)],
                sws[k % 2])

        start_gather(0)
        for k in range(nj):
            if k + 1 < nj:
                if k >= 1:
                    w_wait[k - 1].wait()
                start_gather(k + 1)
            g_wait[k].wait()
            start_write(k)
        w_wait[nj - 2].wait()
        w_wait[nj - 1].wait()

    return sc_gather


def kernel(head_index, rel_type, tail_index, node_emb, node_emb_im, rel_emb):
    batch = head_index.shape[0]
    d = node_emb.shape[1]
    info = plsc.get_sparse_core_info()
    nw = info.num_cores * info.num_subcores
    b_per_w = batch // nw
    n_chunks = b_per_w // CHUNK

    node_cs = _fused_node_table(node_emb.T, node_emb_im.T)
    rel_cs = _trig_table(rel_emb)

    h_idx = head_index.astype(jnp.int32).reshape(nw, n_chunks, CHUNK)
    r_idx = rel_type.astype(jnp.int32).reshape(nw, n_chunks, CHUNK)
    t_idx = tail_index.astype(jnp.int32).reshape(nw, n_chunks, CHUNK)

    sc_gather = _make_sc_gather(batch, 2 * d, nw)
    o_h, o_r, o_t = sc_gather(h_idx, r_idx, t_idx, node_cs, rel_cs)
    return (o_h[:, :d], o_h[:, d:], o_r[:, :d], o_r[:, d:],
            o_t[:, :d], o_t[:, d:])


# speedup vs baseline: 3.9162x; 1.1462x over previous
import functools

import jax
import jax.numpy as jnp
from jax import lax
from jax.experimental import pallas as pl
from jax.experimental.pallas import tpu as pltpu
from jax.experimental.pallas import tpu_sc as plsc

HIDDEN = 64
CHUNK = 128
TBLOCK = 16384


def _fuse_body(a_ref, b_ref, out_ref):
    d2 = 2 * a_ref.shape[0]
    eye = jnp.asarray(
        jax.lax.broadcasted_iota(jnp.int32, (d2, d2), 0)
        == jax.lax.broadcasted_iota(jnp.int32, (d2, d2), 1),
        dtype=jnp.float32)
    ab = jnp.concatenate([a_ref[...], b_ref[...]], axis=0)
    out_ref[...] = jax.lax.dot_general(
        ab, eye, (((0,), (0,)), ((), ())),
        preferred_element_type=jnp.float32)


def _fused_node_table(node_t, node_im_t):
    d, n = node_t.shape
    grid = (n + TBLOCK - 1) // TBLOCK
    return pl.pallas_call(
        _fuse_body,
        grid=(grid,),
        in_specs=[
            pl.BlockSpec((d, TBLOCK), lambda i: (0, i)),
            pl.BlockSpec((d, TBLOCK), lambda i: (0, i)),
        ],
        out_specs=pl.BlockSpec((TBLOCK, 2 * d), lambda i: (i, 0)),
        out_shape=jax.ShapeDtypeStruct((n, 2 * d), jnp.float32),
    )(node_t, node_im_t)


OBLOCK = 4096


def _split_body(h_ref, r_ref, t_ref, hre, him, rre, rim, tre, tim):
    d2 = h_ref.shape[1]
    d = d2 // 2
    eye = jnp.asarray(
        jax.lax.broadcasted_iota(jnp.int32, (d2, d2), 0)
        == jax.lax.broadcasted_iota(jnp.int32, (d2, d2), 1),
        dtype=jnp.float32)
    dn = (((1,), (1,)), ((), ()))
    for ref, (o_re, o_im) in ((h_ref, (hre, him)), (r_ref, (rre, rim)),
                              (t_ref, (tre, tim))):
        ot = jax.lax.dot_general(eye, ref[...], dn,
                                 preferred_element_type=jnp.float32)
        o_re[...] = ot[:d, :]
        o_im[...] = ot[d:, :]


def _split_outputs(o_h, o_r, o_t):
    b, d2 = o_h.shape
    d = d2 // 2
    grid = b // OBLOCK
    in_spec = pl.BlockSpec((OBLOCK, d2), lambda i: (i, 0))
    out_spec = pl.BlockSpec((d, OBLOCK), lambda i: (0, i))
    out_sds = jax.ShapeDtypeStruct((d, b), jnp.float32)
    return pl.pallas_call(
        _split_body,
        grid=(grid,),
        in_specs=[in_spec] * 3,
        out_specs=[out_spec] * 6,
        out_shape=(out_sds,) * 6,
    )(o_h, o_r, o_t)


def _trig_body(rel_ref, cs_ref):
    theta = rel_ref[...]
    cs_ref[:, :HIDDEN] = jnp.cos(theta)
    cs_ref[:, HIDDEN:] = jnp.sin(theta)


def _trig_table(rel_emb):
    r, d = rel_emb.shape
    return pl.pallas_call(
        _trig_body,
        out_shape=jax.ShapeDtypeStruct((r, 2 * d), rel_emb.dtype),
    )(rel_emb)


def _make_sc_gather(batch, d2, nw):
    b_per_w = batch // nw
    n_chunks = b_per_w // CHUNK
    mesh = plsc.VectorSubcoreMesh(core_axis_name="c", subcore_axis_name="s")
    out_sds = jax.ShapeDtypeStruct((batch, d2), jnp.float32)

    @functools.partial(
        pl.kernel,
        mesh=mesh,
        out_type=(out_sds, out_sds, out_sds),
        scratch_types=[
            pltpu.VMEM((n_chunks, CHUNK), jnp.int32),
            pltpu.VMEM((n_chunks, CHUNK), jnp.int32),
            pltpu.VMEM((n_chunks, CHUNK), jnp.int32),
            pltpu.VMEM((CHUNK, d2), jnp.float32),
            pltpu.VMEM((CHUNK, d2), jnp.float32),
            pltpu.SemaphoreType.DMA,
            pltpu.SemaphoreType.DMA,
            pltpu.SemaphoreType.DMA,
            pltpu.SemaphoreType.DMA,
        ],
    )
    def sc_gather(h_idx, r_idx, t_idx, t_node, t_cs,
                  o_h, o_r, o_t,
                  hv, rv, tv, buf0, buf1, sg0, sg1, sw0, sw1):
        nc = 2
        wid = lax.axis_index("s") * nc + lax.axis_index("c")
        base = wid * b_per_w
        pltpu.sync_copy(h_idx.at[wid], hv)
        pltpu.sync_copy(r_idx.at[wid], rv)
        pltpu.sync_copy(t_idx.at[wid], tv)

        bufs = (buf0, buf1)
        sgs = (sg0, sg1)
        sws = (sw0, sw1)
        jobs = [(tab, idx, out, c)
                for (tab, idx, out) in ((t_node, hv, o_h), (t_cs, rv, o_r),
                                        (t_node, tv, o_t))
                for c in range(n_chunks)]
        nj = len(jobs)
        g_wait = [None] * nj
        w_wait = [None] * nj

        def start_gather(k):
            tab, idx, _out, c = jobs[k]
            g_wait[k] = pltpu.async_copy(
                tab.at[idx.at[c]], bufs[k % 2], sgs[k % 2])

        def start_write(k):
            _tab, _idx, out, c = jobs[k]
            w_wait[k] = pltpu.async_copy(
                bufs[k % 2], out.at[pl.ds(base + c * CHUNK, CHUNK)],
                sws[k % 2])

        start_gather(0)
        for k in range(nj):
            if k + 1 < nj:
                if k >= 1:
                    w_wait[k - 1].wait()
                start_gather(k + 1)
            g_wait[k].wait()
            start_write(k)
        w_wait[nj - 2].wait()
        w_wait[nj - 1].wait()

    return sc_gather


def kernel(head_index, rel_type, tail_index, node_emb, node_emb_im, rel_emb):
    batch = head_index.shape[0]
    d = node_emb.shape[1]
    info = plsc.get_sparse_core_info()
    nw = info.num_cores * info.num_subcores
    b_per_w = batch // nw
    n_chunks = b_per_w // CHUNK

    node_cs = _fused_node_table(node_emb.T, node_emb_im.T)
    rel_cs = _trig_table(rel_emb)

    h_idx = head_index.astype(jnp.int32).reshape(nw, n_chunks, CHUNK)
    r_idx = rel_type.astype(jnp.int32).reshape(nw, n_chunks, CHUNK)
    t_idx = tail_index.astype(jnp.int32).reshape(nw, n_chunks, CHUNK)

    sc_gather = _make_sc_gather(batch, 2 * d, nw)
    o_h, o_r, o_t = sc_gather(h_idx, r_idx, t_idx, node_cs, rel_cs)
    outs_t = _split_outputs(o_h, o_r, o_t)
    return tuple(o.T for o in outs_t)
